# baseline (speedup 1.0000x reference)
import functools

import jax
import jax.numpy as jnp
from jax import lax
from jax.experimental import pallas as pl
from jax.experimental.pallas import tpu as pltpu
from jax.experimental.pallas import tpu_sc as plsc

_K_FRAC = 0.15
_MOMENTUM = 0.99998
_HB = 16
_NT = 16
_L = 16


def _loss_kernel(pred_ref, tgt_ref, out_ref):
    x = pred_ref[0]
    t = tgt_ref[0]
    m = jnp.max(x, axis=0)
    s = jnp.sum(jnp.exp(x - m[None, :, :]), axis=0)
    lse = m + jnp.log(s)
    cls = jax.lax.broadcasted_iota(jnp.int32, x.shape, 0)
    tl = jnp.sum(jnp.where(cls == t[None, :, :], x, 0.0), axis=0)
    loss = lse - tl
    out_ref[...] = jnp.where(loss == 0.0, 0.0, loss)


def _select_kernel(num, n_per_tile, loss_hbm, out_hbm, data_v, histf_v,
                   tileh_v, tmp_v, part_v, out_v, shared_hist, shared_part):
    sid = lax.axis_index("s")
    nv = n_per_tile // _L
    iota = lax.broadcasted_iota(jnp.int32, (_L,), 0)
    ones = jnp.ones((_L,), jnp.int32)
    pltpu.sync_copy(loss_hbm.at[pl.ds(sid * n_per_tile, n_per_tile)], data_v)

    prefix = jnp.int32(0)
    k_rem = jnp.int32(num)
    for p, (shift, nbits) in enumerate(((20, 11), (10, 10), (0, 10))):
        nbins = 1 << nbits
        nch = nbins // _L

        def zero_body(j, _):
            for l in range(_L):
                histf_v[pl.ds(l * 2048 + j * _L, _L)] = jnp.zeros((_L,), jnp.int32)
            return 0

        lax.fori_loop(0, nch, zero_body, 0)

        def hist_body(j, _, shift=shift, nbins=nbins, p=p, prefix=prefix):
            v = data_v[pl.ds(j * _L, _L)]
            e = lax.bitcast_convert_type(v, jnp.int32)
            b = jnp.right_shift(e, jnp.int32(shift)) & jnp.int32(nbins - 1)
            fidx = iota * jnp.int32(2048) + b
            if p == 0:
                plsc.addupdate_scatter(histf_v, [fidx], ones)
            else:
                hi = jnp.right_shift(e, jnp.int32(shift + nbits))
                active = hi == prefix
                plsc.addupdate_scatter(histf_v, [fidx], ones, mask=active)
            return 0

        lax.fori_loop(0, nv, hist_body, 0)

        def merge_body(j, _):
            acc = histf_v[pl.ds(j * _L, _L)]
            for l in range(1, _L):
                acc = acc + histf_v[pl.ds(l * 2048 + j * _L, _L)]
            tileh_v[pl.ds(j * _L, _L)] = acc
            return 0

        lax.fori_loop(0, nch, merge_body, 0)

        pltpu.sync_copy(tileh_v.at[pl.ds(0, nbins)],
                        shared_hist.at[pl.ds(sid * 2048, nbins)])
        plsc.subcore_barrier()
        pltpu.sync_copy(shared_hist, histf_v)
        plsc.subcore_barrier()

        def scan_body(i, carry):
            acc, w, found, ksub = carry
            c = nch - 1 - i
            chunk = histf_v[pl.ds(c * _L, _L)]
            for l in range(1, _NT):
                chunk = chunk + histf_v[pl.ds(l * 2048 + c * _L, _L)]
            suf = lax.rev(jnp.cumsum(lax.rev(chunk, (0,))), (0,)) + acc
            cond = suf >= k_rem
            t = jnp.sum(cond.astype(jnp.int32)) - 1
            has = t >= 0
            sel = jnp.logical_and(has, jnp.logical_not(found))
            lane_eq = iota == t
            s_w = jnp.sum(jnp.where(lane_eq, suf, 0))
            h_w = jnp.sum(jnp.where(lane_eq, chunk, 0))
            w = jnp.where(sel, c * _L + t, w)
            ksub = jnp.where(sel, s_w - h_w, ksub)
            found = jnp.logical_or(found, has)
            acc = acc + jnp.sum(chunk)
            return acc, w, found, ksub

        _, w, _, ksub = lax.fori_loop(
            0, nch, scan_body,
            (jnp.int32(0), jnp.int32(0), jnp.bool_(False), jnp.int32(0)))
        prefix = jnp.left_shift(prefix, jnp.int32(nbits)) | w
        k_rem = k_rem - ksub

    thr = prefix

    def sum_body(j, carry):
        svec, cvec = carry
        v = data_v[pl.ds(j * _L, _L)]
        e = lax.bitcast_convert_type(v, jnp.int32)
        m = e >= thr
        svec = svec + jnp.where(m, v, 0.0)
        cvec = cvec + jnp.where(m, jnp.int32(1), jnp.int32(0))
        return svec, cvec

    svec, cvec = lax.fori_loop(
        0, nv, sum_body,
        (jnp.zeros((_L,), jnp.float32), jnp.zeros((_L,), jnp.int32)))
    psum = jnp.sum(svec)
    pcnt = jnp.sum(cvec).astype(jnp.float32)

    tmp_v[...] = jnp.where(iota == 0, psum,
                           jnp.where(iota == 1, pcnt, 0.0))
    pltpu.sync_copy(tmp_v, shared_part.at[pl.ds(sid * _L, _L)])
    plsc.subcore_barrier()

    @pl.when(sid == 0)
    def _emit():
        pltpu.sync_copy(shared_part, part_v)
        acc = part_v[pl.ds(0, _L)]
        for l in range(1, _NT):
            acc = acc + part_v[pl.ds(l * _L, _L)]
        ts = jnp.sum(jnp.where(iota == 0, acc, 0.0))
        tc = jnp.sum(jnp.where(iota == 1, acc, 0.0))
        tsv = jnp.zeros((_L,), jnp.float32) + ts
        tcv = jnp.zeros((_L,), jnp.float32) + tc
        out_v[...] = tsv / tcv
        pltpu.sync_copy(out_v, out_hbm)


def kernel(pred, target, step):
    B, C, H, W = pred.shape
    num = int(_K_FRAC * B * H * W * max(_MOMENTUM ** 1000, _K_FRAC))
    tgt = target.astype(jnp.int32)
    grid = (B, H // _HB)
    loss = pl.pallas_call(
        _loss_kernel,
        grid=grid,
        in_specs=[
            pl.BlockSpec((1, C, _HB, W), lambda b, h: (b, 0, h, 0)),
            pl.BlockSpec((1, _HB, W), lambda b, h: (b, h, 0)),
        ],
        out_specs=pl.BlockSpec((_HB, W), lambda b, h: (b * (H // _HB) + h, 0)),
        out_shape=jax.ShapeDtypeStruct((B * H, W), jnp.float32),
        compiler_params=pltpu.CompilerParams(
            dimension_semantics=("arbitrary", "arbitrary"),
        ),
    )(pred, tgt)

    n = B * H * W
    n_per_tile = n // _NT
    mesh = plsc.VectorSubcoreMesh(
        core_axis_name="c", subcore_axis_name="s", num_cores=1)
    sel = functools.partial(
        pl.kernel,
        mesh=mesh,
        compiler_params=pltpu.CompilerParams(needs_layout_passes=False),
        out_type=jax.ShapeDtypeStruct((_L,), jnp.float32),
        scratch_types=[
            pltpu.VMEM((n_per_tile,), jnp.float32),
            pltpu.VMEM((_L * 2048,), jnp.int32),
            pltpu.VMEM((2048,), jnp.int32),
            pltpu.VMEM((_L,), jnp.float32),
            pltpu.VMEM((_NT * _L,), jnp.float32),
            pltpu.VMEM((_L,), jnp.float32),
            pltpu.VMEM_SHARED((_NT * 2048,), jnp.int32),
            pltpu.VMEM_SHARED((_NT * _L,), jnp.float32),
        ],
    )(functools.partial(_select_kernel, num, n_per_tile))
    out = sel(loss.reshape(-1))
    return out[0]
